# Initial kernel scaffold; baseline (speedup 1.0000x reference)
import functools

import jax
import jax.numpy as jnp
from jax import lax
from jax.experimental import pallas as pl
from jax.experimental.pallas import tpu as pltpu
from jax.experimental.pallas import tpu_sc as plsc

N = 10000
D = 256
STEPS = 3
HALF = N // 2
NSUB = 16
CH = 128
NCHK = 79
EPT = NCHK * CH
EPAD = EPT * NSUB
ACC_R = 5120
DUMMY = 5100
ZROWS = ACC_R // NSUB
DRAIN = 320
RB = 1000


def _k0_body(x_ref, wembT, bemb, weT, whhT, bhh, h_ref, m_ref, gh_ref):
    h = jnp.maximum(
        jnp.dot(x_ref[...], wembT[...], preferred_element_type=jnp.float32)
        + bemb[...], 0.0)
    h_ref[...] = h
    m_ref[...] = jnp.dot(h, weT[...], preferred_element_type=jnp.float32)
    gh_ref[...] = (
        jnp.dot(h, whhT[...], preferred_element_type=jnp.float32) + bhh[...])


def _gru(a, h, gh, wihT, bih):
    gx = jnp.dot(a, wihT, preferred_element_type=jnp.float32) + bih
    r = jax.nn.sigmoid(gx[:, :D] + gh[:, :D])
    z = jax.nn.sigmoid(gx[:, D:2 * D] + gh[:, D:2 * D])
    n = jnp.tanh(gx[:, 2 * D:] + r * gh[:, 2 * D:])
    return (1.0 - z) * n + z * h


def _k1_body(a_ref, h_ref, gh_ref, wihT, bih, weT, whhT, bhh,
             hn_ref, mn_ref, ghn_ref):
    hnew = _gru(a_ref[...], h_ref[...], gh_ref[...], wihT[...], bih[...])
    hn_ref[...] = hnew
    mn_ref[...] = jnp.dot(hnew, weT[...], preferred_element_type=jnp.float32)
    ghn_ref[...] = (
        jnp.dot(hnew, whhT[...], preferred_element_type=jnp.float32)
        + bhh[...])


def _k2_body(a_ref, h_ref, gh_ref, wihT, bih, woutT, bout, out_ref):
    hnew = _gru(a_ref[...], h_ref[...], gh_ref[...], wihT[...], bih[...])
    out_ref[...] = jnp.tanh(
        jnp.dot(hnew, woutT[...], preferred_element_type=jnp.float32)
        + bout[...])


def _row_spec(cols):
    return pl.BlockSpec((RB, cols), lambda i: (i, 0))


def _full_spec(r, c):
    return pl.BlockSpec((r, c), lambda i: (0, 0))


def _mk_k0():
    return pl.pallas_call(
        _k0_body,
        grid=(N // RB,),
        in_specs=[
            _row_spec(D), _full_spec(D, D), _full_spec(1, D),
            _full_spec(D, D), _full_spec(D, 3 * D), _full_spec(1, 3 * D),
        ],
        out_specs=[_row_spec(D), _row_spec(D), _row_spec(3 * D)],
        out_shape=[
            jax.ShapeDtypeStruct((N, D), jnp.float32),
            jax.ShapeDtypeStruct((N, D), jnp.float32),
            jax.ShapeDtypeStruct((N, 3 * D), jnp.float32),
        ],
    )


def _mk_k1():
    return pl.pallas_call(
        _k1_body,
        grid=(N // RB,),
        in_specs=[
            _row_spec(D), _row_spec(D), _row_spec(3 * D),
            _full_spec(D, 3 * D), _full_spec(1, 3 * D),
            _full_spec(D, D), _full_spec(D, 3 * D), _full_spec(1, 3 * D),
        ],
        out_specs=[_row_spec(D), _row_spec(D), _row_spec(3 * D)],
        out_shape=[
            jax.ShapeDtypeStruct((N, D), jnp.float32),
            jax.ShapeDtypeStruct((N, D), jnp.float32),
            jax.ShapeDtypeStruct((N, 3 * D), jnp.float32),
        ],
    )


def _mk_k2():
    return pl.pallas_call(
        _k2_body,
        grid=(N // RB,),
        in_specs=[
            _row_spec(D), _row_spec(D), _row_spec(3 * D),
            _full_spec(D, 3 * D), _full_spec(1, 3 * D),
            _full_spec(D, D), _full_spec(1, D),
        ],
        out_specs=_row_spec(D),
        out_shape=jax.ShapeDtypeStruct((N, D), jnp.float32),
    )


def _sc_segment_sum(m, srcr, dstr, zrows):
    @functools.partial(
        pl.kernel,
        out_type=jax.ShapeDtypeStruct((N, D), jnp.float32),
        mesh=plsc.VectorSubcoreMesh(core_axis_name="c", subcore_axis_name="s"),
        scratch_types=[
            pltpu.VMEM((CH,), jnp.int32),
            pltpu.VMEM((CH,), jnp.int32),
            pltpu.VMEM((CH, D), jnp.float32),
            pltpu.VMEM_SHARED((ACC_R, D), jnp.float32),
            pltpu.SemaphoreType.DMA,
        ],
    )
    def sc_k(m_hbm, src_hbm, dst_hbm, z_hbm, a_hbm,
             idx_v, dst_v, rows_v, acc_sh, sem):
        c = lax.axis_index("c")
        s = lax.axis_index("s")

        pltpu.sync_copy(z_hbm, rows_v)
        base = s * ZROWS
        pltpu.sync_copy(rows_v, acc_sh.at[pl.ds(base, CH)])
        pltpu.sync_copy(rows_v, acc_sh.at[pl.ds(base + CH, CH)])
        pltpu.sync_copy(rows_v.at[pl.ds(0, 64)],
                        acc_sh.at[pl.ds(base + 2 * CH, 64)])
        plsc.subcore_barrier()

        def body(j, carry):
            row = s * NCHK + j
            pltpu.sync_copy(src_hbm.at[row], idx_v)
            pltpu.sync_copy(dst_hbm.at[c, row], dst_v)
            pltpu.async_copy(m_hbm.at[idx_v], rows_v, sem).wait()
            pltpu.sync_copy(rows_v, acc_sh.at[dst_v], add=True)
            return carry

        lax.fori_loop(0, NCHK, body, 0)
        plsc.subcore_barrier()

        out_base = c * HALF + s * DRAIN

        @pl.when(s < NSUB - 1)
        def _():
            pltpu.sync_copy(acc_sh.at[pl.ds(s * DRAIN, DRAIN)],
                            a_hbm.at[pl.ds(out_base, DRAIN)])

        @pl.when(s == NSUB - 1)
        def _():
            pltpu.sync_copy(acc_sh.at[pl.ds(s * DRAIN, HALF - 15 * DRAIN)],
                            a_hbm.at[pl.ds(out_base, HALF - 15 * DRAIN)])

    return sc_k(m, srcr, dstr, zrows)


def kernel(x, edge_index, W_emb, b_emb, W_e, W_ih, b_ih, W_hh, b_hh,
           W_out, b_out):
    src = edge_index[0].astype(jnp.int32)
    dst = edge_index[1].astype(jnp.int32)
    e = src.shape[0]
    pad = EPAD - e
    src_p = jnp.concatenate([src, jnp.zeros((pad,), jnp.int32)])
    dst0 = jnp.concatenate(
        [jnp.where(dst < HALF, dst, DUMMY),
         jnp.full((pad,), DUMMY, jnp.int32)])
    dst1 = jnp.concatenate(
        [jnp.where(dst >= HALF, dst - HALF, DUMMY),
         jnp.full((pad,), DUMMY, jnp.int32)])
    srcr = src_p.reshape(NSUB * NCHK, CH)
    dstr = jnp.stack([dst0, dst1]).reshape(2, NSUB * NCHK, CH)
    zrows = jnp.zeros((CH, D), jnp.float32)

    wembT = W_emb.T
    weT = W_e.T
    wihT = W_ih.T
    whhT = W_hh.T
    woutT = W_out.T
    bemb = b_emb.reshape(1, D)
    bih = b_ih.reshape(1, 3 * D)
    bhh = b_hh.reshape(1, 3 * D)
    bout = b_out.reshape(1, D)

    h, m, gh = _mk_k0()(x, wembT, bemb, weT, whhT, bhh)
    out = None
    for step in range(STEPS):
        a = _sc_segment_sum(m, srcr, dstr, zrows)
        if step < STEPS - 1:
            h, m, gh = _mk_k1()(a, h, gh, wihT, bih, weT, whhT, bhh)
        else:
            out = _mk_k2()(a, h, gh, wihT, bih, woutT, bout)
    return out


# speedup vs baseline: 3.1508x; 3.1508x over previous
import functools

import jax
import jax.numpy as jnp
from jax import lax
from jax.experimental import pallas as pl
from jax.experimental.pallas import tpu as pltpu
from jax.experimental.pallas import tpu_sc as plsc

N = 10000
D = 256
DH = 128
STEPS = 3
NSUB = 16
CH = 128
NCHK = 79
ACC_R = 10240
DUMMY = 10000
ZROWS = ACC_R // NSUB
DRAIN = 640
RB = 1000


def _k0_body(x_ref, wembT, bemb, weT, whhT, bhh, h_ref, m0_ref, m1_ref,
             gh_ref):
    h = jnp.maximum(
        jnp.dot(x_ref[...], wembT[...], preferred_element_type=jnp.float32)
        + bemb[...], 0.0)
    h_ref[...] = h
    m = jnp.dot(h, weT[...], preferred_element_type=jnp.float32)
    m0_ref[...] = m[:, :DH]
    m1_ref[...] = m[:, DH:]
    gh_ref[...] = (
        jnp.dot(h, whhT[...], preferred_element_type=jnp.float32) + bhh[...])


def _gru(a0, a1, h, gh, wihT, bih):
    gx = (jnp.dot(a0, wihT[:DH], preferred_element_type=jnp.float32)
          + jnp.dot(a1, wihT[DH:], preferred_element_type=jnp.float32)
          + bih)
    r = jax.nn.sigmoid(gx[:, :D] + gh[:, :D])
    z = jax.nn.sigmoid(gx[:, D:2 * D] + gh[:, D:2 * D])
    n = jnp.tanh(gx[:, 2 * D:] + r * gh[:, 2 * D:])
    return (1.0 - z) * n + z * h


def _k1_body(a0_ref, a1_ref, h_ref, gh_ref, wihT, bih, weT, whhT, bhh,
             hn_ref, mn0_ref, mn1_ref, ghn_ref):
    hnew = _gru(a0_ref[0], a1_ref[0], h_ref[...], gh_ref[...], wihT[...],
                bih[...])
    hn_ref[...] = hnew
    m = jnp.dot(hnew, weT[...], preferred_element_type=jnp.float32)
    mn0_ref[...] = m[:, :DH]
    mn1_ref[...] = m[:, DH:]
    ghn_ref[...] = (
        jnp.dot(hnew, whhT[...], preferred_element_type=jnp.float32)
        + bhh[...])


def _k2_body(a0_ref, a1_ref, h_ref, gh_ref, wihT, bih, woutT, bout, out_ref):
    hnew = _gru(a0_ref[0], a1_ref[0], h_ref[...], gh_ref[...], wihT[...],
                bih[...])
    out_ref[...] = jnp.tanh(
        jnp.dot(hnew, woutT[...], preferred_element_type=jnp.float32)
        + bout[...])


def _row_spec(cols):
    return pl.BlockSpec((RB, cols), lambda i: (i, 0))


def _half_spec(which):
    return pl.BlockSpec((1, RB, DH), lambda i: (which, i, 0))


def _full_spec(r, c):
    return pl.BlockSpec((r, c), lambda i: (0, 0))


def _mk_k0():
    return pl.pallas_call(
        _k0_body,
        grid=(N // RB,),
        in_specs=[
            _row_spec(D), _full_spec(D, D), _full_spec(1, D),
            _full_spec(D, D), _full_spec(D, 3 * D), _full_spec(1, 3 * D),
        ],
        out_specs=[_row_spec(D), _row_spec(DH), _row_spec(DH),
                   _row_spec(3 * D)],
        out_shape=[
            jax.ShapeDtypeStruct((N, D), jnp.float32),
            jax.ShapeDtypeStruct((N, DH), jnp.float32),
            jax.ShapeDtypeStruct((N, DH), jnp.float32),
            jax.ShapeDtypeStruct((N, 3 * D), jnp.float32),
        ],
    )


def _mk_k1():
    return pl.pallas_call(
        _k1_body,
        grid=(N // RB,),
        in_specs=[
            _half_spec(0), _half_spec(1), _row_spec(D), _row_spec(3 * D),
            _full_spec(D, 3 * D), _full_spec(1, 3 * D),
            _full_spec(D, D), _full_spec(D, 3 * D), _full_spec(1, 3 * D),
        ],
        out_specs=[_row_spec(D), _row_spec(DH), _row_spec(DH),
                   _row_spec(3 * D)],
        out_shape=[
            jax.ShapeDtypeStruct((N, D), jnp.float32),
            jax.ShapeDtypeStruct((N, DH), jnp.float32),
            jax.ShapeDtypeStruct((N, DH), jnp.float32),
            jax.ShapeDtypeStruct((N, 3 * D), jnp.float32),
        ],
    )


def _mk_k2():
    return pl.pallas_call(
        _k2_body,
        grid=(N // RB,),
        in_specs=[
            _half_spec(0), _half_spec(1), _row_spec(D), _row_spec(3 * D),
            _full_spec(D, 3 * D), _full_spec(1, 3 * D),
            _full_spec(D, D), _full_spec(1, D),
        ],
        out_specs=_row_spec(D),
        out_shape=jax.ShapeDtypeStruct((N, D), jnp.float32),
    )


def _sc_segment_sum(m0, m1, srcr, dstr, zrows):
    @functools.partial(
        pl.kernel,
        out_type=jax.ShapeDtypeStruct((2, N, DH), jnp.float32),
        mesh=plsc.VectorSubcoreMesh(core_axis_name="c", subcore_axis_name="s"),
        scratch_types=[
            pltpu.VMEM((CH,), jnp.int32),
            pltpu.VMEM((CH,), jnp.int32),
            pltpu.VMEM((CH, DH), jnp.float32),
            pltpu.VMEM_SHARED((ACC_R, DH), jnp.float32),
            pltpu.SemaphoreType.DMA,
        ],
    )
    def sc_k(m0_hbm, m1_hbm, src_hbm, dst_hbm, z_hbm, a_hbm,
             idx_v, dst_v, rows_v, acc_sh, sem):
        c = lax.axis_index("c")
        s = lax.axis_index("s")

        pltpu.sync_copy(z_hbm, rows_v)
        base = s * ZROWS
        for t in range(ZROWS // CH):
            pltpu.sync_copy(rows_v, acc_sh.at[pl.ds(base + t * CH, CH)])
        plsc.subcore_barrier()

        def body(j, carry):
            row = s * NCHK + j
            pltpu.sync_copy(src_hbm.at[row], idx_v)
            pltpu.sync_copy(dst_hbm.at[row], dst_v)

            @pl.when(c == 0)
            def _():
                pltpu.async_copy(m0_hbm.at[idx_v], rows_v, sem).wait()

            @pl.when(c == 1)
            def _():
                pltpu.async_copy(m1_hbm.at[idx_v], rows_v, sem).wait()

            pltpu.sync_copy(rows_v, acc_sh.at[dst_v], add=True)
            return carry

        lax.fori_loop(0, NCHK, body, 0)
        plsc.subcore_barrier()

        out_base = s * DRAIN

        @pl.when(s < NSUB - 1)
        def _():
            pltpu.sync_copy(acc_sh.at[pl.ds(out_base, DRAIN)],
                            a_hbm.at[c, pl.ds(out_base, DRAIN)])

        @pl.when(s == NSUB - 1)
        def _():
            pltpu.sync_copy(acc_sh.at[pl.ds(out_base, N - 15 * DRAIN)],
                            a_hbm.at[c, pl.ds(out_base, N - 15 * DRAIN)])

    return sc_k(m0, m1, srcr, dstr, zrows)


def kernel(x, edge_index, W_emb, b_emb, W_e, W_ih, b_ih, W_hh, b_hh,
           W_out, b_out):
    src = edge_index[0].astype(jnp.int32)
    dst = edge_index[1].astype(jnp.int32)
    e = src.shape[0]
    epad = NSUB * NCHK * CH
    pad = epad - e
    srcr = jnp.concatenate(
        [src, jnp.zeros((pad,), jnp.int32)]).reshape(NSUB * NCHK, CH)
    dstr = jnp.concatenate(
        [dst, jnp.full((pad,), DUMMY, jnp.int32)]).reshape(NSUB * NCHK, CH)
    zrows = jnp.zeros((CH, DH), jnp.float32)

    wembT = W_emb.T
    weT = W_e.T
    wihT = W_ih.T
    whhT = W_hh.T
    woutT = W_out.T
    bemb = b_emb.reshape(1, D)
    bih = b_ih.reshape(1, 3 * D)
    bhh = b_hh.reshape(1, 3 * D)
    bout = b_out.reshape(1, D)

    h, m0, m1, gh = _mk_k0()(x, wembT, bemb, weT, whhT, bhh)
    out = None
    for step in range(STEPS):
        a = _sc_segment_sum(m0, m1, srcr, dstr, zrows)
        if step < STEPS - 1:
            h, m0, m1, gh = _mk_k1()(a, a, h, gh, wihT, bih, weT, whhT, bhh)
        else:
            out = _mk_k2()(a, a, h, gh, wihT, bih, woutT, bout)
    return out


# speedup vs baseline: 3.1712x; 1.0065x over previous
import functools

import jax
import jax.numpy as jnp
from jax import lax
from jax.experimental import pallas as pl
from jax.experimental.pallas import tpu as pltpu
from jax.experimental.pallas import tpu_sc as plsc

N = 10000
D = 256
DH = 128
STEPS = 3
NSUB = 16
CH = 128
NCHK = 80
IBLK = 16
NBLK = NCHK // IBLK
ACC_R = 10112
DUMMY = 10000
ZROWS = ACC_R // NSUB
DRAIN = 632
RB = 1000


def _k0_body(x_ref, wembT, bemb, weT, whhT, bhh, h_ref, m0_ref, m1_ref,
             gh_ref):
    h = jnp.maximum(
        jnp.dot(x_ref[...], wembT[...], preferred_element_type=jnp.float32)
        + bemb[...], 0.0)
    h_ref[...] = h
    m = jnp.dot(h, weT[...], preferred_element_type=jnp.float32)
    m0_ref[...] = m[:, :DH]
    m1_ref[...] = m[:, DH:]
    gh_ref[...] = (
        jnp.dot(h, whhT[...], preferred_element_type=jnp.float32) + bhh[...])


def _gru(a0, a1, h, gh, wihT, bih):
    gx = (jnp.dot(a0, wihT[:DH], preferred_element_type=jnp.float32)
          + jnp.dot(a1, wihT[DH:], preferred_element_type=jnp.float32)
          + bih)
    r = jax.nn.sigmoid(gx[:, :D] + gh[:, :D])
    z = jax.nn.sigmoid(gx[:, D:2 * D] + gh[:, D:2 * D])
    n = jnp.tanh(gx[:, 2 * D:] + r * gh[:, 2 * D:])
    return (1.0 - z) * n + z * h


def _k1_body(a0_ref, a1_ref, h_ref, gh_ref, wihT, bih, weT, whhT, bhh,
             hn_ref, mn0_ref, mn1_ref, ghn_ref):
    hnew = _gru(a0_ref[0], a1_ref[0], h_ref[...], gh_ref[...], wihT[...],
                bih[...])
    hn_ref[...] = hnew
    m = jnp.dot(hnew, weT[...], preferred_element_type=jnp.float32)
    mn0_ref[...] = m[:, :DH]
    mn1_ref[...] = m[:, DH:]
    ghn_ref[...] = (
        jnp.dot(hnew, whhT[...], preferred_element_type=jnp.float32)
        + bhh[...])


def _k2_body(a0_ref, a1_ref, h_ref, gh_ref, wihT, bih, woutT, bout, out_ref):
    hnew = _gru(a0_ref[0], a1_ref[0], h_ref[...], gh_ref[...], wihT[...],
                bih[...])
    out_ref[...] = jnp.tanh(
        jnp.dot(hnew, woutT[...], preferred_element_type=jnp.float32)
        + bout[...])


def _row_spec(cols):
    return pl.BlockSpec((RB, cols), lambda i: (i, 0))


def _half_spec(which):
    return pl.BlockSpec((1, RB, DH), lambda i: (which, i, 0))


def _full_spec(r, c):
    return pl.BlockSpec((r, c), lambda i: (0, 0))


def _mk_k0():
    return pl.pallas_call(
        _k0_body,
        grid=(N // RB,),
        in_specs=[
            _row_spec(D), _full_spec(D, D), _full_spec(1, D),
            _full_spec(D, D), _full_spec(D, 3 * D), _full_spec(1, 3 * D),
        ],
        out_specs=[_row_spec(D), _row_spec(DH), _row_spec(DH),
                   _row_spec(3 * D)],
        out_shape=[
            jax.ShapeDtypeStruct((N, D), jnp.float32),
            jax.ShapeDtypeStruct((N, DH), jnp.float32),
            jax.ShapeDtypeStruct((N, DH), jnp.float32),
            jax.ShapeDtypeStruct((N, 3 * D), jnp.float32),
        ],
    )


def _mk_k1():
    return pl.pallas_call(
        _k1_body,
        grid=(N // RB,),
        in_specs=[
            _half_spec(0), _half_spec(1), _row_spec(D), _row_spec(3 * D),
            _full_spec(D, 3 * D), _full_spec(1, 3 * D),
            _full_spec(D, D), _full_spec(D, 3 * D), _full_spec(1, 3 * D),
        ],
        out_specs=[_row_spec(D), _row_spec(DH), _row_spec(DH),
                   _row_spec(3 * D)],
        out_shape=[
            jax.ShapeDtypeStruct((N, D), jnp.float32),
            jax.ShapeDtypeStruct((N, DH), jnp.float32),
            jax.ShapeDtypeStruct((N, DH), jnp.float32),
            jax.ShapeDtypeStruct((N, 3 * D), jnp.float32),
        ],
    )


def _mk_k2():
    return pl.pallas_call(
        _k2_body,
        grid=(N // RB,),
        in_specs=[
            _half_spec(0), _half_spec(1), _row_spec(D), _row_spec(3 * D),
            _full_spec(D, 3 * D), _full_spec(1, 3 * D),
            _full_spec(D, D), _full_spec(1, D),
        ],
        out_specs=_row_spec(D),
        out_shape=jax.ShapeDtypeStruct((N, D), jnp.float32),
    )


def _sc_segment_sum(m0, m1, srcr, dstr, zrows):
    @functools.partial(
        pl.kernel,
        out_type=jax.ShapeDtypeStruct((2, N, DH), jnp.float32),
        mesh=plsc.VectorSubcoreMesh(core_axis_name="c", subcore_axis_name="s"),
        scratch_types=[
            pltpu.VMEM((IBLK, CH), jnp.int32),
            pltpu.VMEM((IBLK, CH), jnp.int32),
            pltpu.VMEM((CH, DH), jnp.float32),
            pltpu.VMEM((CH, DH), jnp.float32),
            pltpu.VMEM_SHARED((ACC_R, DH), jnp.float32),
            pltpu.SemaphoreType.DMA,
            pltpu.SemaphoreType.DMA,
        ],
    )
    def sc_k(m0_hbm, m1_hbm, src_hbm, dst_hbm, z_hbm, a_hbm,
             srcs_v, dsts_v, rows0_v, rows1_v, acc_sh, sem0, sem1):
        c = lax.axis_index("c")
        s = lax.axis_index("s")
        rows = [rows0_v, rows1_v]
        sems = [sem0, sem1]

        pltpu.sync_copy(z_hbm, acc_sh.at[pl.ds(s * ZROWS, ZROWS)])
        plsc.subcore_barrier()

        def run(mh):
            def blk_body(blk, carry):
                row0 = s * NCHK + blk * IBLK
                pltpu.sync_copy(src_hbm.at[pl.ds(row0, IBLK)], srcs_v)
                pltpu.sync_copy(dst_hbm.at[pl.ds(row0, IBLK)], dsts_v)
                pltpu.async_copy(mh.at[srcs_v.at[0]], rows[0], sems[0])
                pltpu.async_copy(mh.at[srcs_v.at[1]], rows[1], sems[1])

                def body(t, cy):
                    for b in range(2):
                        chunk = 2 * t + b
                        pltpu.make_async_copy(
                            mh.at[srcs_v.at[chunk]], rows[b], sems[b]).wait()
                        pltpu.sync_copy(rows[b],
                                        acc_sh.at[dsts_v.at[chunk]],
                                        add=True)

                        @pl.when(chunk + 2 < IBLK)
                        def _():
                            pltpu.async_copy(mh.at[srcs_v.at[chunk + 2]],
                                             rows[b], sems[b])
                    return cy

                lax.fori_loop(0, IBLK // 2, body, 0, unroll=False)
                return carry

            lax.fori_loop(0, NBLK, blk_body, 0, unroll=False)

        @pl.when(c == 0)
        def _():
            run(m0_hbm)

        @pl.when(c == 1)
        def _():
            run(m1_hbm)

        plsc.subcore_barrier()

        out_base = s * DRAIN

        @pl.when(s < NSUB - 1)
        def _():
            pltpu.sync_copy(acc_sh.at[pl.ds(out_base, DRAIN)],
                            a_hbm.at[c, pl.ds(out_base, DRAIN)])

        @pl.when(s == NSUB - 1)
        def _():
            pltpu.sync_copy(acc_sh.at[pl.ds(out_base, N - 15 * DRAIN)],
                            a_hbm.at[c, pl.ds(out_base, N - 15 * DRAIN)])

    return sc_k(m0, m1, srcr, dstr, zrows)


def kernel(x, edge_index, W_emb, b_emb, W_e, W_ih, b_ih, W_hh, b_hh,
           W_out, b_out):
    src = edge_index[0].astype(jnp.int32)
    dst = edge_index[1].astype(jnp.int32)
    e = src.shape[0]
    epad = NSUB * NCHK * CH
    pad = epad - e
    srcr = jnp.concatenate(
        [src, jnp.zeros((pad,), jnp.int32)]).reshape(NSUB * NCHK, CH)
    dstr = jnp.concatenate(
        [dst, jnp.full((pad,), DUMMY, jnp.int32)]).reshape(NSUB * NCHK, CH)
    zrows = jnp.zeros((ZROWS, DH), jnp.float32)

    wembT = W_emb.T
    weT = W_e.T
    wihT = W_ih.T
    whhT = W_hh.T
    woutT = W_out.T
    bemb = b_emb.reshape(1, D)
    bih = b_ih.reshape(1, 3 * D)
    bhh = b_hh.reshape(1, 3 * D)
    bout = b_out.reshape(1, D)

    h, m0, m1, gh = _mk_k0()(x, wembT, bemb, weT, whhT, bhh)
    out = None
    for step in range(STEPS):
        a = _sc_segment_sum(m0, m1, srcr, dstr, zrows)
        if step < STEPS - 1:
            h, m0, m1, gh = _mk_k1()(a, a, h, gh, wihT, bih, weT, whhT, bhh)
        else:
            out = _mk_k2()(a, a, h, gh, wihT, bih, woutT, bout)
    return out


# speedup vs baseline: 3.2275x; 1.0178x over previous
import functools

import jax
import jax.numpy as jnp
from jax import lax
from jax.experimental import pallas as pl
from jax.experimental.pallas import tpu as pltpu
from jax.experimental.pallas import tpu_sc as plsc

N = 10000
D = 256
DH = 128
STEPS = 3
NSUB = 16
CH = 128
NCHK = 80
NB = 2
IBLK = 40
NBLK = NCHK // IBLK
ACC_R = 10112
DUMMY = 10000
ZROWS = ACC_R // NSUB
DRAIN = 632
RB = 1000


def _k0_body(x_ref, wembT, bemb, weT, whhT, bhh, h_ref, m0_ref, m1_ref,
             gh_ref):
    h = jnp.maximum(
        jnp.dot(x_ref[...], wembT[...], preferred_element_type=jnp.float32)
        + bemb[...], 0.0)
    h_ref[...] = h
    m = jnp.dot(h, weT[...], preferred_element_type=jnp.float32)
    m0_ref[...] = m[:, :DH]
    m1_ref[...] = m[:, DH:]
    gh_ref[...] = (
        jnp.dot(h, whhT[...], preferred_element_type=jnp.float32) + bhh[...])


def _gru(a0, a1, h, gh, wihT, bih):
    gx = (jnp.dot(a0, wihT[:DH], preferred_element_type=jnp.float32)
          + jnp.dot(a1, wihT[DH:], preferred_element_type=jnp.float32)
          + bih)
    r = jax.nn.sigmoid(gx[:, :D] + gh[:, :D])
    z = jax.nn.sigmoid(gx[:, D:2 * D] + gh[:, D:2 * D])
    n = jnp.tanh(gx[:, 2 * D:] + r * gh[:, 2 * D:])
    return (1.0 - z) * n + z * h


def _k1_body(a0_ref, a1_ref, h_ref, gh_ref, wihT, bih, weT, whhT, bhh,
             hn_ref, mn0_ref, mn1_ref, ghn_ref):
    hnew = _gru(a0_ref[0], a1_ref[0], h_ref[...], gh_ref[...], wihT[...],
                bih[...])
    hn_ref[...] = hnew
    m = jnp.dot(hnew, weT[...], preferred_element_type=jnp.float32)
    mn0_ref[...] = m[:, :DH]
    mn1_ref[...] = m[:, DH:]
    ghn_ref[...] = (
        jnp.dot(hnew, whhT[...], preferred_element_type=jnp.float32)
        + bhh[...])


def _k2_body(a0_ref, a1_ref, h_ref, gh_ref, wihT, bih, woutT, bout, out_ref):
    hnew = _gru(a0_ref[0], a1_ref[0], h_ref[...], gh_ref[...], wihT[...],
                bih[...])
    out_ref[...] = jnp.tanh(
        jnp.dot(hnew, woutT[...], preferred_element_type=jnp.float32)
        + bout[...])


def _row_spec(cols):
    return pl.BlockSpec((RB, cols), lambda i: (i, 0))


def _half_spec(which):
    return pl.BlockSpec((1, RB, DH), lambda i: (which, i, 0))


def _full_spec(r, c):
    return pl.BlockSpec((r, c), lambda i: (0, 0))


def _mk_k0():
    return pl.pallas_call(
        _k0_body,
        grid=(N // RB,),
        in_specs=[
            _row_spec(D), _full_spec(D, D), _full_spec(1, D),
            _full_spec(D, D), _full_spec(D, 3 * D), _full_spec(1, 3 * D),
        ],
        out_specs=[_row_spec(D), _row_spec(DH), _row_spec(DH),
                   _row_spec(3 * D)],
        out_shape=[
            jax.ShapeDtypeStruct((N, D), jnp.float32),
            jax.ShapeDtypeStruct((N, DH), jnp.float32),
            jax.ShapeDtypeStruct((N, DH), jnp.float32),
            jax.ShapeDtypeStruct((N, 3 * D), jnp.float32),
        ],
    )


def _mk_k1():
    return pl.pallas_call(
        _k1_body,
        grid=(N // RB,),
        in_specs=[
            _half_spec(0), _half_spec(1), _row_spec(D), _row_spec(3 * D),
            _full_spec(D, 3 * D), _full_spec(1, 3 * D),
            _full_spec(D, D), _full_spec(D, 3 * D), _full_spec(1, 3 * D),
        ],
        out_specs=[_row_spec(D), _row_spec(DH), _row_spec(DH),
                   _row_spec(3 * D)],
        out_shape=[
            jax.ShapeDtypeStruct((N, D), jnp.float32),
            jax.ShapeDtypeStruct((N, DH), jnp.float32),
            jax.ShapeDtypeStruct((N, DH), jnp.float32),
            jax.ShapeDtypeStruct((N, 3 * D), jnp.float32),
        ],
    )


def _mk_k2():
    return pl.pallas_call(
        _k2_body,
        grid=(N // RB,),
        in_specs=[
            _half_spec(0), _half_spec(1), _row_spec(D), _row_spec(3 * D),
            _full_spec(D, 3 * D), _full_spec(1, 3 * D),
            _full_spec(D, D), _full_spec(1, D),
        ],
        out_specs=_row_spec(D),
        out_shape=jax.ShapeDtypeStruct((N, D), jnp.float32),
    )


def _sc_segment_sum(m0, m1, srcr, dstr, zrows):
    @functools.partial(
        pl.kernel,
        out_type=jax.ShapeDtypeStruct((2, N, DH), jnp.float32),
        mesh=plsc.VectorSubcoreMesh(core_axis_name="c", subcore_axis_name="s"),
        scratch_types=[
            pltpu.VMEM((IBLK, CH), jnp.int32),
            pltpu.VMEM((IBLK, CH), jnp.int32),
            pltpu.VMEM((CH, DH), jnp.float32),
            pltpu.VMEM((CH, DH), jnp.float32),
            pltpu.VMEM_SHARED((ACC_R, DH), jnp.float32),
            pltpu.SemaphoreType.DMA,
            pltpu.SemaphoreType.DMA,
            pltpu.SemaphoreType.DMA,
            pltpu.SemaphoreType.DMA,
        ],
    )
    def sc_k(m0_hbm, m1_hbm, src_hbm, dst_hbm, z_hbm, a_hbm,
             srcs_v, dsts_v, r0, r1, acc_sh, g0, g1, s0, s1):
        c = lax.axis_index("c")
        s = lax.axis_index("s")
        rows = [r0, r1]
        gsem = [g0, g1]
        ssem = [s0, s1]

        pltpu.sync_copy(z_hbm, acc_sh.at[pl.ds(s * ZROWS, ZROWS)])
        plsc.subcore_barrier()

        def run(mh):
            def blk_body(blk, carry):
                row0 = s * NCHK + blk * IBLK
                pltpu.sync_copy(src_hbm.at[pl.ds(row0, IBLK)], srcs_v)
                pltpu.sync_copy(dst_hbm.at[pl.ds(row0, IBLK)], dsts_v)
                pltpu.async_copy(mh.at[srcs_v.at[0]], rows[0], gsem[0])
                pltpu.async_copy(mh.at[srcs_v.at[1]], rows[1], gsem[1])

                def body(t, cy):
                    for b in range(2):
                        chunk = 2 * t + b
                        pltpu.make_async_copy(
                            mh.at[srcs_v.at[chunk]], rows[b], gsem[b]).wait()
                        pltpu.async_copy(rows[b],
                                         acc_sh.at[dsts_v.at[chunk]],
                                         ssem[b], add=True)

                        @pl.when(chunk + 2 < IBLK)
                        def _():
                            pltpu.make_async_copy(
                                rows[b], acc_sh.at[dsts_v.at[chunk]],
                                ssem[b]).wait()
                            pltpu.async_copy(mh.at[srcs_v.at[chunk + 2]],
                                             rows[b], gsem[b])
                    return cy

                lax.fori_loop(0, IBLK // 2, body, 0, unroll=False)
                pltpu.make_async_copy(rows[0], acc_sh.at[dsts_v.at[IBLK - 2]],
                                      ssem[0]).wait()
                pltpu.make_async_copy(rows[1], acc_sh.at[dsts_v.at[IBLK - 1]],
                                      ssem[1]).wait()
                return carry

            lax.fori_loop(0, NBLK, blk_body, 0, unroll=False)

        @pl.when(c == 0)
        def _():
            run(m0_hbm)

        @pl.when(c == 1)
        def _():
            run(m1_hbm)

        plsc.subcore_barrier()

        out_base = s * DRAIN

        @pl.when(s < NSUB - 1)
        def _():
            pltpu.sync_copy(acc_sh.at[pl.ds(out_base, DRAIN)],
                            a_hbm.at[c, pl.ds(out_base, DRAIN)])

        @pl.when(s == NSUB - 1)
        def _():
            pltpu.sync_copy(acc_sh.at[pl.ds(out_base, N - 15 * DRAIN)],
                            a_hbm.at[c, pl.ds(out_base, N - 15 * DRAIN)])

    return sc_k(m0, m1, srcr, dstr, zrows)


def kernel(x, edge_index, W_emb, b_emb, W_e, W_ih, b_ih, W_hh, b_hh,
           W_out, b_out):
    src = edge_index[0].astype(jnp.int32)
    dst = edge_index[1].astype(jnp.int32)
    e = src.shape[0]
    epad = NSUB * NCHK * CH
    pad = epad - e
    srcr = jnp.concatenate(
        [src, jnp.zeros((pad,), jnp.int32)]).reshape(NSUB * NCHK, CH)
    dstr = jnp.concatenate(
        [dst, jnp.full((pad,), DUMMY, jnp.int32)]).reshape(NSUB * NCHK, CH)
    zrows = jnp.zeros((ZROWS, DH), jnp.float32)

    wembT = W_emb.T
    weT = W_e.T
    wihT = W_ih.T
    whhT = W_hh.T
    woutT = W_out.T
    bemb = b_emb.reshape(1, D)
    bih = b_ih.reshape(1, 3 * D)
    bhh = b_hh.reshape(1, 3 * D)
    bout = b_out.reshape(1, D)

    h, m0, m1, gh = _mk_k0()(x, wembT, bemb, weT, whhT, bhh)
    out = None
    for step in range(STEPS):
        a = _sc_segment_sum(m0, m1, srcr, dstr, zrows)
        if step < STEPS - 1:
            h, m0, m1, gh = _mk_k1()(a, a, h, gh, wihT, bih, weT, whhT, bhh)
        else:
            out = _mk_k2()(a, a, h, gh, wihT, bih, woutT, bout)
    return out
